# baseline (speedup 1.0000x reference)
import functools

import jax
import jax.numpy as jnp
from jax import lax
from jax.experimental import pallas as pl
from jax.experimental.pallas import tpu as pltpu
from jax.experimental.pallas import tpu_sc as plsc

_K = 5


def _sc_lookup(src_table, tgt_table, src_idx, tgt_idx, L):
    V, D = src_table.shape
    N = src_idx.shape[0]
    B = N // L
    n_outer = L // _K

    mesh = plsc.VectorSubcoreMesh(core_axis_name="c", subcore_axis_name="s")
    info = plsc.get_sparse_core_info()
    NC = info.num_cores
    NW = NC * info.num_subcores
    W = B // NW

    @functools.partial(
        pl.kernel,
        mesh=mesh,
        out_type=[
            jax.ShapeDtypeStruct((N, D), jnp.float32),
            jax.ShapeDtypeStruct((N, D), jnp.float32),
        ],
        scratch_types=[
            pltpu.VMEM((L * W,), jnp.int32),
            pltpu.VMEM((L * W,), jnp.int32),
            pltpu.VMEM((_K * W, D), jnp.float32),
            pltpu.SemaphoreType.DMA((_K,)),
            pltpu.SemaphoreType.DMA((_K,)),
        ],
    )
    def k(src_t, tgt_t, src_i, tgt_i, src_o, tgt_o,
          idx_src_v, idx_tgt_v, rows_v, sem_g, sem_w):
        wid = lax.axis_index("s") * NC + lax.axis_index("c")
        col = wid * W
        base = wid * L * W

        pltpu.sync_copy(src_i.at[pl.ds(base, L * W)], idx_src_v)
        pltpu.sync_copy(tgt_i.at[pl.ds(base, L * W)], idx_tgt_v)

        def wb_wait(out_hbm, b):
            pltpu.make_async_copy(
                rows_v.at[pl.ds(0, W)],
                out_hbm.at[pl.ds(base, W)],
                sem_w.at[b],
            ).wait()

        def side(table, idx_v, out_hbm, prev_out):
            def outer(t, carry):
                gathers = []
                for b in range(_K):
                    @pl.when(t > 0)
                    def _():
                        wb_wait(out_hbm, b)

                    if prev_out is not None:
                        @pl.when(t == 0)
                        def _():
                            wb_wait(prev_out, b)

                    l = t * _K + b
                    gathers.append(pltpu.async_copy(
                        table.at[idx_v.at[pl.ds(l * W, W)]],
                        rows_v.at[pl.ds(b * W, W)],
                        sem_g.at[b]))

                for b in range(_K):
                    gathers[b].wait()
                    l = t * _K + b
                    pltpu.async_copy(
                        rows_v.at[pl.ds(b * W, W)],
                        out_hbm.at[pl.ds(base + l * W, W)],
                        sem_w.at[b])
                return carry

            lax.fori_loop(0, n_outer, outer, 0)

        side(src_t, idx_src_v, src_o, None)
        side(tgt_t, idx_tgt_v, tgt_o, src_o)

        for b in range(_K):
            wb_wait(tgt_o, b)

    return k(src_table, tgt_table, src_idx, tgt_idx)


def kernel(src_table, tgt_table, src_indices, tgt_indices):
    B, L = src_indices.shape
    D = src_table.shape[1]
    src_out, tgt_out = _sc_lookup(
        src_table, tgt_table,
        src_indices.astype(jnp.int32).T.reshape(-1),
        tgt_indices.astype(jnp.int32).T.reshape(-1), L)
    src_out = src_out.reshape(L, B, D).transpose(1, 0, 2)
    tgt_out = tgt_out.reshape(L, B, D).transpose(1, 0, 2)
    return (src_out, tgt_out)
